# Initial kernel scaffold; baseline (speedup 1.0000x reference)
import functools

import jax
import jax.numpy as jnp
from jax import lax
from jax.experimental import pallas as pl
from jax.experimental.pallas import tpu as pltpu
from jax.experimental.pallas import tpu_sc as plsc

N = 10000
E = 320000
D_IN = 128
H = 64
L = 4

W = 80
NC = 2
NS = 16
NW = NC * NS
CB = 128
CH = 79
ET = CH * CB
EP = NW * ET
NP = 10240
RPT = NP // NS

BE = 2048
BN = 1024

_mesh = plsc.VectorSubcoreMesh(
    core_axis_name="c", subcore_axis_name="s", num_cores=NC, num_subcores=NS)


def _silu(x):
    return x * jax.nn.sigmoid(x)


@functools.partial(
    pl.kernel,
    out_type=(jax.ShapeDtypeStruct((EP, W), jnp.float32),
              jax.ShapeDtypeStruct((EP, W), jnp.float32)),
    mesh=_mesh,
    scratch_types=[
        pltpu.VMEM((CH, CB), jnp.int32),
        pltpu.VMEM((CH, CB), jnp.int32),
        pltpu.VMEM((CB, W), jnp.float32),
        pltpu.VMEM((CB, W), jnp.float32),
        pltpu.SemaphoreType.DMA,
        pltpu.SemaphoreType.DMA,
    ],
)
def _sc_gather(ts_hbm, tr_hbm, sg_hbm, rg_hbm, es_hbm, er_hbm,
               sidx, ridx, buf_s, buf_r, sem_s, sem_r):
    c = lax.axis_index("c")
    t = lax.axis_index("s")
    wid = t * NC + c
    pltpu.sync_copy(sg_hbm.at[wid], sidx)
    pltpu.sync_copy(rg_hbm.at[wid], ridx)
    ebase = wid * ET

    def body(j, carry):
        cs = pltpu.async_copy(ts_hbm.at[sidx.at[j]], buf_s, sem_s)
        cr = pltpu.async_copy(tr_hbm.at[ridx.at[j]], buf_r, sem_r)
        cs.wait()
        cr.wait()
        row0 = ebase + j * CB
        pltpu.sync_copy(buf_s, es_hbm.at[pl.ds(row0, CB)])
        pltpu.sync_copy(buf_r, er_hbm.at[pl.ds(row0, CB)])
        return carry

    lax.fori_loop(0, CH, body, 0)


@functools.partial(
    pl.kernel,
    out_type=(jax.ShapeDtypeStruct((NP, W), jnp.float32),
              jax.ShapeDtypeStruct((NP, W), jnp.float32)),
    mesh=_mesh,
    scratch_types=[
        pltpu.VMEM((CH, CB), jnp.int32),
        pltpu.VMEM((CB, W), jnp.float32),
        pltpu.VMEM((CB, W), jnp.float32),
        pltpu.VMEM_SHARED((NP, W), jnp.float32),
    ],
)
def _sc_scatter(eo_hbm, rs_hbm, p0_hbm, p1_hbm, ridx, zb, eb, acc):
    c = lax.axis_index("c")
    t = lax.axis_index("s")
    wid = t * NC + c
    pltpu.sync_copy(rs_hbm.at[wid], ridx)

    def zrow(i, carry):
        for k in range(W // 16):
            zb[i, pl.ds(k * 16, 16)] = jnp.zeros((16,), jnp.float32)
        return carry
    lax.fori_loop(0, CB, zrow, 0)
    for q in range(RPT // CB):
        pltpu.sync_copy(zb, acc.at[pl.ds(t * RPT + q * CB, CB)])
    plsc.subcore_barrier()

    ebase = wid * ET

    def body(j, carry):
        pltpu.sync_copy(eo_hbm.at[pl.ds(ebase + j * CB, CB)], eb)
        pltpu.sync_copy(eb, acc.at[ridx.at[j]], add=True)
        return carry
    lax.fori_loop(0, CH, body, 0)
    plsc.subcore_barrier()

    for q in range(RPT // CB):
        row0 = t * RPT + q * CB
        pltpu.sync_copy(acc.at[pl.ds(row0, CB)], eb)

        @pl.when(c == 0)
        def _():
            pltpu.sync_copy(eb, p0_hbm.at[pl.ds(row0, CB)])

        @pl.when(c == 1)
        def _():
            pltpu.sync_copy(eb, p1_hbm.at[pl.ds(row0, CB)])


def _embed_body(x_ref, pos_ref, win_ref, bin_ref, ws_ref, be1_ref, wr_ref,
                h_ref, ts_ref, tr_ref):
    xb = x_ref[...]
    pb = pos_ref[...]
    h0 = xb @ win_ref[...] + bin_ref[...]
    hs = h0 @ ws_ref[...] + be1_ref[...]
    hr = h0 @ wr_ref[...]
    z = jnp.zeros((xb.shape[0], W - H - 4), jnp.float32)
    h_ref[...] = h0
    ts_ref[...] = jnp.concatenate([hs, pb, z], axis=1)
    tr_ref[...] = jnp.concatenate([hr, -pb, z], axis=1)


def _edge_body(es_ref, er_ref, q_ref, we2_ref, be2_ref, wp1_ref, bp1_ref,
               wp2_ref, bp2_ref, eo_ref):
    f = es_ref[...] + er_ref[...]
    m1 = _silu(f[:, :H] + (f * f) @ q_ref[...])
    m = _silu(m1 @ we2_ref[...] + be2_ref[...])
    q1 = _silu(m @ wp1_ref[...] + bp1_ref[...])
    pw = q1 @ wp2_ref[...] + bp2_ref[...]
    d16 = f[:, H:W]
    t16 = jnp.clip(d16 * pw, -100.0, 100.0)
    eo_ref[...] = jnp.concatenate([m, t16], axis=1)


def _node_body(h_ref, pos_ref, p0_ref, p1_ref, wn1a_ref, wn1b_ref, bn1_ref,
               wn2_ref, bn2_ref, wsn_ref, be1n_ref, wrn_ref,
               hn_ref, posn_ref, ts_ref, tr_ref):
    h = h_ref[...]
    p = p0_ref[...] + p1_ref[...]
    u1 = _silu(h @ wn1a_ref[...] + p @ wn1b_ref[...] + bn1_ref[...])
    hn = h + u1 @ wn2_ref[...] + bn2_ref[...]
    posn = pos_ref[...] + p[:, H:H + 4]
    hs = hn @ wsn_ref[...] + be1n_ref[...]
    hr = hn @ wrn_ref[...]
    z = jnp.zeros((h.shape[0], W - H - 4), jnp.float32)
    hn_ref[...] = hn
    posn_ref[...] = posn
    ts_ref[...] = jnp.concatenate([hs, posn, z], axis=1)
    tr_ref[...] = jnp.concatenate([hr, -posn, z], axis=1)


def _node_last_body(h_ref, pos_ref, p0_ref, p1_ref, wn1a_ref, wn1b_ref,
                    bn1_ref, wn2_ref, bn2_ref, wout_ref, bout_ref,
                    out_ref, posn_ref):
    h = h_ref[...]
    p = p0_ref[...] + p1_ref[...]
    u1 = _silu(h @ wn1a_ref[...] + p @ wn1b_ref[...] + bn1_ref[...])
    hn = h + u1 @ wn2_ref[...] + bn2_ref[...]
    out_ref[...] = hn @ wout_ref[...] + bout_ref[...]
    posn_ref[...] = pos_ref[...] + p[:, H:H + 4]


def _full(shape):
    return pl.BlockSpec(shape, lambda i: (0,) * len(shape))


def _rows(b, w):
    return pl.BlockSpec((b, w), lambda i: (i, 0))


def _tc_embed(xp, posp, w_in, b_in, ws0, be10, wr0):
    return pl.pallas_call(
        _embed_body,
        grid=(NP // BN,),
        in_specs=[_rows(BN, D_IN), _rows(BN, 4), _full((D_IN, H)),
                  _full((1, H)), _full((H, H)), _full((1, H)), _full((H, H))],
        out_specs=[_rows(BN, H), _rows(BN, W), _rows(BN, W)],
        out_shape=[jax.ShapeDtypeStruct((NP, H), jnp.float32),
                   jax.ShapeDtypeStruct((NP, W), jnp.float32),
                   jax.ShapeDtypeStruct((NP, W), jnp.float32)],
    )(xp, posp, w_in, b_in, ws0, be10, wr0)


def _tc_edge(es, er, q, we2, be2, wp1, bp1, wp2, bp2):
    return pl.pallas_call(
        _edge_body,
        grid=(EP // BE,),
        in_specs=[_rows(BE, W), _rows(BE, W), _full((W, H)), _full((H, H)),
                  _full((1, H)), _full((H, H)), _full((1, H)), _full((H, 1)),
                  _full((1, 1))],
        out_specs=[_rows(BE, W)],
        out_shape=[jax.ShapeDtypeStruct((EP, W), jnp.float32)],
    )(es, er, q, we2, be2, wp1, bp1, wp2, bp2)[0]


def _tc_node(h, posp, p0, p1, wn1a, wn1b, bn1, wn2, bn2, wsn, be1n, wrn):
    return pl.pallas_call(
        _node_body,
        grid=(NP // BN,),
        in_specs=[_rows(BN, H), _rows(BN, 4), _rows(BN, W), _rows(BN, W),
                  _full((H, H)), _full((W, H)), _full((1, H)), _full((H, H)),
                  _full((1, H)), _full((H, H)), _full((1, H)), _full((H, H))],
        out_specs=[_rows(BN, H), _rows(BN, 4), _rows(BN, W), _rows(BN, W)],
        out_shape=[jax.ShapeDtypeStruct((NP, H), jnp.float32),
                   jax.ShapeDtypeStruct((NP, 4), jnp.float32),
                   jax.ShapeDtypeStruct((NP, W), jnp.float32),
                   jax.ShapeDtypeStruct((NP, W), jnp.float32)],
    )(h, posp, p0, p1, wn1a, wn1b, bn1, wn2, bn2, wsn, be1n, wrn)


def _tc_node_last(h, posp, p0, p1, wn1a, wn1b, bn1, wn2, bn2, w_out, b_out):
    return pl.pallas_call(
        _node_last_body,
        grid=(NP // BN,),
        in_specs=[_rows(BN, H), _rows(BN, 4), _rows(BN, W), _rows(BN, W),
                  _full((H, H)), _full((W, H)), _full((1, H)), _full((H, H)),
                  _full((1, H)), _full((H, H)), _full((1, H))],
        out_specs=[_rows(BN, H), _rows(BN, 4)],
        out_shape=[jax.ShapeDtypeStruct((NP, H), jnp.float32),
                   jax.ShapeDtypeStruct((NP, 4), jnp.float32)],
    )(h, posp, p0, p1, wn1a, wn1b, bn1, wn2, bn2, w_out, b_out)


def kernel(x, pos, edge_index, W_in, b_in, We1, be1, We2, be2, Wp1, bp1,
           Wp2, bp2, Wn1, bn1, Wn2, bn2, W_out, b_out):
    f32 = jnp.float32
    s = edge_index[0].astype(jnp.int32)
    r = edge_index[1].astype(jnp.int32)
    padg = jnp.zeros((EP - E,), jnp.int32)
    sg = jnp.concatenate([s, padg]).reshape(NW, CH, CB)
    rg = jnp.concatenate([r, padg]).reshape(NW, CH, CB)
    rs = jnp.concatenate(
        [r, jnp.full((EP - E,), NP - 1, jnp.int32)]).reshape(NW, CH, CB)

    xp = jnp.zeros((NP, D_IN), f32).at[:N].set(x)
    posp = jnp.zeros((NP, 4), f32).at[:N, :3].set(pos)

    ws = [We1[l][:H] for l in range(L)]
    wr = [We1[l][H:2 * H] for l in range(L)]
    qm = [jnp.zeros((W, H), f32).at[H:H + 3, :].set(
        jnp.broadcast_to(We1[l][2 * H], (3, H))) for l in range(L)]
    wn1a = [Wn1[l][:H] for l in range(L)]
    wn1b = [jnp.zeros((W, H), f32).at[:H].set(Wn1[l][H:]) for l in range(L)]
    r2 = lambda v: v.reshape(1, -1)

    h, ts, tr = _tc_embed(xp, posp, W_in, r2(b_in), ws[0], r2(be1[0]), wr[0])
    for l in range(L):
        es, er = _sc_gather(ts, tr, sg, rg)
        eo = _tc_edge(es, er, qm[l], We2[l], r2(be2[l]), Wp1[l], r2(bp1[l]),
                      Wp2[l], r2(bp2[l]))
        p0, p1 = _sc_scatter(eo, rs)
        if l < L - 1:
            h, posp, ts, tr = _tc_node(
                h, posp, p0, p1, wn1a[l], wn1b[l], r2(bn1[l]), Wn2[l],
                r2(bn2[l]), ws[l + 1], r2(be1[l + 1]), wr[l + 1])
        else:
            out, posp = _tc_node_last(
                h, posp, p0, p1, wn1a[l], wn1b[l], r2(bn1[l]), Wn2[l],
                r2(bn2[l]), W_out, r2(b_out))
    return (out[:N], posp[:N, :3])


# speedup vs baseline: 4.0931x; 4.0931x over previous
import functools

import jax
import jax.numpy as jnp
from jax import lax
from jax.experimental import pallas as pl
from jax.experimental.pallas import tpu as pltpu
from jax.experimental.pallas import tpu_sc as plsc

N = 10000
E = 320000
D_IN = 128
H = 64
L = 4

W = 128
NC = 2
NS = 16
NW = NC * NS
CB = 128
CH = 79
ET = CH * CB
EP = NW * ET
NP = 10240
RPT = NP // NS

BE = 2048
BN = 1024

def _silu(x):
    return x * jax.nn.sigmoid(x)


def _gather_body(ts_hbm, tr_hbm, sg_hbm, rg_hbm, es_hbm, er_hbm,
                 sidx, ridx, buf_s, buf_r, sem_s, sem_r):
    c = lax.axis_index("c")
    t = lax.axis_index("s")
    wid = t * NC + c
    pltpu.sync_copy(sg_hbm.at[wid], sidx)
    pltpu.sync_copy(rg_hbm.at[wid], ridx)
    ebase = wid * ET

    def body(j, carry):
        cs = pltpu.async_copy(ts_hbm.at[sidx.at[j]], buf_s, sem_s)
        cr = pltpu.async_copy(tr_hbm.at[ridx.at[j]], buf_r, sem_r)
        cs.wait()
        cr.wait()
        row0 = ebase + j * CB
        pltpu.sync_copy(buf_s, es_hbm.at[pl.ds(row0, CB)])
        pltpu.sync_copy(buf_r, er_hbm.at[pl.ds(row0, CB)])
        return carry

    lax.fori_loop(0, CH, body, 0)


def _scatter_body(eo_hbm, rs_hbm, p0_hbm, p1_hbm, ridx, zb, eb, acc):
    c = lax.axis_index("c")
    t = lax.axis_index("s")
    wid = t * NC + c
    pltpu.sync_copy(rs_hbm.at[wid], ridx)

    def zrow(i, carry):
        for k in range(W // 16):
            zb[i, pl.ds(k * 16, 16)] = jnp.zeros((16,), jnp.float32)
        return carry
    lax.fori_loop(0, CB, zrow, 0)
    for q in range(RPT // CB):
        pltpu.sync_copy(zb, acc.at[pl.ds(t * RPT + q * CB, CB)])
    plsc.subcore_barrier()

    ebase = wid * ET

    def body(j, carry):
        pltpu.sync_copy(eo_hbm.at[pl.ds(ebase + j * CB, CB)], eb)
        pltpu.sync_copy(eb, acc.at[ridx.at[j]], add=True)
        return carry
    lax.fori_loop(0, CH, body, 0)
    plsc.subcore_barrier()

    for q in range(RPT // CB):
        row0 = t * RPT + q * CB
        pltpu.sync_copy(acc.at[pl.ds(row0, CB)], eb)

        @pl.when(c == 0)
        def _():
            pltpu.sync_copy(eb, p0_hbm.at[pl.ds(row0, CB)])

        @pl.when(c == 1)
        def _():
            pltpu.sync_copy(eb, p1_hbm.at[pl.ds(row0, CB)])


@functools.lru_cache(maxsize=1)
def _sc_kernels():
    mesh = plsc.VectorSubcoreMesh(
        core_axis_name="c", subcore_axis_name="s",
        num_cores=NC, num_subcores=NS)
    gather = pl.kernel(
        _gather_body,
        out_type=(jax.ShapeDtypeStruct((EP, W), jnp.float32),
                  jax.ShapeDtypeStruct((EP, W), jnp.float32)),
        mesh=mesh,
        scratch_types=[
            pltpu.VMEM((CH, CB), jnp.int32),
            pltpu.VMEM((CH, CB), jnp.int32),
            pltpu.VMEM((CB, W), jnp.float32),
            pltpu.VMEM((CB, W), jnp.float32),
            pltpu.SemaphoreType.DMA,
            pltpu.SemaphoreType.DMA,
        ],
    )
    scatter = pl.kernel(
        _scatter_body,
        out_type=(jax.ShapeDtypeStruct((NP, W), jnp.float32),
                  jax.ShapeDtypeStruct((NP, W), jnp.float32)),
        mesh=mesh,
        scratch_types=[
            pltpu.VMEM((CH, CB), jnp.int32),
            pltpu.VMEM((CB, W), jnp.float32),
            pltpu.VMEM((CB, W), jnp.float32),
            pltpu.VMEM_SHARED((NP, W), jnp.float32),
        ],
    )
    return gather, scatter


def _embed_body(x_ref, pos_ref, win_ref, bin_ref, ws_ref, be1_ref, wr_ref,
                h_ref, ts_ref, tr_ref):
    xb = x_ref[...]
    pb = pos_ref[...]
    h0 = xb @ win_ref[...] + bin_ref[...]
    hs = h0 @ ws_ref[...] + be1_ref[...]
    hr = h0 @ wr_ref[...]
    z = jnp.zeros((xb.shape[0], W - H - 4), jnp.float32)
    h_ref[...] = h0
    ts_ref[...] = jnp.concatenate([hs, pb, z], axis=1)
    tr_ref[...] = jnp.concatenate([hr, -pb, z], axis=1)


def _edge_body(es_ref, er_ref, q_ref, we2_ref, be2_ref, wp1_ref, bp1_ref,
               wp2_ref, bp2_ref, eo_ref):
    f = es_ref[...] + er_ref[...]
    m1 = _silu(f[:, :H] + (f * f) @ q_ref[...])
    m = _silu(m1 @ we2_ref[...] + be2_ref[...])
    q1 = _silu(m @ wp1_ref[...] + bp1_ref[...])
    pw = q1 @ wp2_ref[...] + bp2_ref[...]
    d16 = f[:, H:W]
    t16 = jnp.clip(d16 * pw, -100.0, 100.0)
    eo_ref[...] = jnp.concatenate([m, t16], axis=1)


def _node_body(h_ref, pos_ref, p0_ref, p1_ref, wn1a_ref, wn1b_ref, bn1_ref,
               wn2_ref, bn2_ref, wsn_ref, be1n_ref, wrn_ref,
               hn_ref, posn_ref, ts_ref, tr_ref):
    h = h_ref[...]
    p = p0_ref[...] + p1_ref[...]
    u1 = _silu(h @ wn1a_ref[...] + p @ wn1b_ref[...] + bn1_ref[...])
    hn = h + u1 @ wn2_ref[...] + bn2_ref[...]
    posn = pos_ref[...] + p[:, H:H + 4]
    hs = hn @ wsn_ref[...] + be1n_ref[...]
    hr = hn @ wrn_ref[...]
    z = jnp.zeros((h.shape[0], W - H - 4), jnp.float32)
    hn_ref[...] = hn
    posn_ref[...] = posn
    ts_ref[...] = jnp.concatenate([hs, posn, z], axis=1)
    tr_ref[...] = jnp.concatenate([hr, -posn, z], axis=1)


def _node_last_body(h_ref, pos_ref, p0_ref, p1_ref, wn1a_ref, wn1b_ref,
                    bn1_ref, wn2_ref, bn2_ref, wout_ref, bout_ref,
                    out_ref, posn_ref):
    h = h_ref[...]
    p = p0_ref[...] + p1_ref[...]
    u1 = _silu(h @ wn1a_ref[...] + p @ wn1b_ref[...] + bn1_ref[...])
    hn = h + u1 @ wn2_ref[...] + bn2_ref[...]
    out_ref[...] = hn @ wout_ref[...] + bout_ref[...]
    posn_ref[...] = pos_ref[...] + p[:, H:H + 4]


def _full(shape):
    return pl.BlockSpec(shape, lambda i: (0,) * len(shape))


def _rows(b, w):
    return pl.BlockSpec((b, w), lambda i: (i, 0))


def _tc_embed(xp, posp, w_in, b_in, ws0, be10, wr0):
    return pl.pallas_call(
        _embed_body,
        grid=(NP // BN,),
        in_specs=[_rows(BN, D_IN), _rows(BN, 4), _full((D_IN, H)),
                  _full((1, H)), _full((H, H)), _full((1, H)), _full((H, H))],
        out_specs=[_rows(BN, H), _rows(BN, W), _rows(BN, W)],
        out_shape=[jax.ShapeDtypeStruct((NP, H), jnp.float32),
                   jax.ShapeDtypeStruct((NP, W), jnp.float32),
                   jax.ShapeDtypeStruct((NP, W), jnp.float32)],
    )(xp, posp, w_in, b_in, ws0, be10, wr0)


def _tc_edge(es, er, q, we2, be2, wp1, bp1, wp2, bp2):
    return pl.pallas_call(
        _edge_body,
        grid=(EP // BE,),
        in_specs=[_rows(BE, W), _rows(BE, W), _full((W, H)), _full((H, H)),
                  _full((1, H)), _full((H, H)), _full((1, H)), _full((H, 1)),
                  _full((1, 1))],
        out_specs=[_rows(BE, W)],
        out_shape=[jax.ShapeDtypeStruct((EP, W), jnp.float32)],
    )(es, er, q, we2, be2, wp1, bp1, wp2, bp2)[0]


def _tc_node(h, posp, p0, p1, wn1a, wn1b, bn1, wn2, bn2, wsn, be1n, wrn):
    return pl.pallas_call(
        _node_body,
        grid=(NP // BN,),
        in_specs=[_rows(BN, H), _rows(BN, 4), _rows(BN, W), _rows(BN, W),
                  _full((H, H)), _full((W, H)), _full((1, H)), _full((H, H)),
                  _full((1, H)), _full((H, H)), _full((1, H)), _full((H, H))],
        out_specs=[_rows(BN, H), _rows(BN, 4), _rows(BN, W), _rows(BN, W)],
        out_shape=[jax.ShapeDtypeStruct((NP, H), jnp.float32),
                   jax.ShapeDtypeStruct((NP, 4), jnp.float32),
                   jax.ShapeDtypeStruct((NP, W), jnp.float32),
                   jax.ShapeDtypeStruct((NP, W), jnp.float32)],
    )(h, posp, p0, p1, wn1a, wn1b, bn1, wn2, bn2, wsn, be1n, wrn)


def _tc_node_last(h, posp, p0, p1, wn1a, wn1b, bn1, wn2, bn2, w_out, b_out):
    return pl.pallas_call(
        _node_last_body,
        grid=(NP // BN,),
        in_specs=[_rows(BN, H), _rows(BN, 4), _rows(BN, W), _rows(BN, W),
                  _full((H, H)), _full((W, H)), _full((1, H)), _full((H, H)),
                  _full((1, H)), _full((H, H)), _full((1, H))],
        out_specs=[_rows(BN, H), _rows(BN, 4)],
        out_shape=[jax.ShapeDtypeStruct((NP, H), jnp.float32),
                   jax.ShapeDtypeStruct((NP, 4), jnp.float32)],
    )(h, posp, p0, p1, wn1a, wn1b, bn1, wn2, bn2, w_out, b_out)


def kernel(x, pos, edge_index, W_in, b_in, We1, be1, We2, be2, Wp1, bp1,
           Wp2, bp2, Wn1, bn1, Wn2, bn2, W_out, b_out):
    f32 = jnp.float32
    s = edge_index[0].astype(jnp.int32)
    r = edge_index[1].astype(jnp.int32)
    padg = jnp.zeros((EP - E,), jnp.int32)
    sg = jnp.concatenate([s, padg]).reshape(NW, CH, CB)
    rg = jnp.concatenate([r, padg]).reshape(NW, CH, CB)
    rs = jnp.concatenate(
        [r, jnp.full((EP - E,), NP - 1, jnp.int32)]).reshape(NW, CH, CB)

    xp = jnp.zeros((NP, D_IN), f32).at[:N].set(x)
    posp = jnp.zeros((NP, 4), f32).at[:N, :3].set(pos)

    ws = [We1[l][:H] for l in range(L)]
    wr = [We1[l][H:2 * H] for l in range(L)]
    qm = [jnp.zeros((W, H), f32).at[H:H + 3, :].set(
        jnp.broadcast_to(We1[l][2 * H], (3, H))) for l in range(L)]
    wn1a = [Wn1[l][:H] for l in range(L)]
    wn1b = [jnp.zeros((W, H), f32).at[:H].set(Wn1[l][H:]) for l in range(L)]
    r2 = lambda v: v.reshape(1, -1)

    sc_gather, sc_scatter = _sc_kernels()
    h, ts, tr = _tc_embed(xp, posp, W_in, r2(b_in), ws[0], r2(be1[0]), wr[0])
    for l in range(L):
        es, er = sc_gather(ts, tr, sg, rg)
        eo = _tc_edge(es, er, qm[l], We2[l], r2(be2[l]), Wp1[l], r2(bp1[l]),
                      Wp2[l], r2(bp2[l]))
        p0, p1 = sc_scatter(eo, rs)
        if l < L - 1:
            h, posp, ts, tr = _tc_node(
                h, posp, p0, p1, wn1a[l], wn1b[l], r2(bn1[l]), Wn2[l],
                r2(bn2[l]), ws[l + 1], r2(be1[l + 1]), wr[l + 1])
        else:
            out, posp = _tc_node_last(
                h, posp, p0, p1, wn1a[l], wn1b[l], r2(bn1[l]), Wn2[l],
                r2(bn2[l]), W_out, r2(b_out))
    return (out[:N], posp[:N, :3])
